# baseline (speedup 1.0000x reference)
import jax
import jax.numpy as jnp
from jax.experimental import pallas as pl
from jax.experimental.pallas import tpu as pltpu

_BLK = 8192
_UNROLL = 128


def _gather_kernel(ids_ref, table_ref, out_hbm, smem_a, smem_b, slab_a,
                   slab_b, sem_ids, sems):
    i = pl.program_id(0)
    nb = pl.num_programs(0)
    blk = slab_a.shape[0]

    def out_copy(slab, step, slot):
        return pltpu.make_async_copy(
            slab.at[:, 0],
            out_hbm.at[pl.ds(step * blk, blk), :],
            sems.at[slot])

    def ids_copy(step, smem, slot):
        return pltpu.make_async_copy(
            ids_ref.at[step, 0], smem.at[0], sem_ids.at[slot])

    @pl.when(i == 0)
    def _():
        ids_copy(0, smem_a, 0).start()

    def run(slab, smem, smem_next, slot):
        @pl.when(i + 1 < nb)
        def _():
            ids_copy(i + 1, smem_next, 1 - slot).start()

        ids_copy(i, smem, slot).wait()

        @pl.when(i >= 2)
        def _():
            out_copy(slab, i - 2, slot).wait()

        def body(o, _):
            base = o * _UNROLL
            for k in range(_UNROLL):
                slab[base + k, 0] = table_ref[smem[0, base + k], 0]
            return 0

        jax.lax.fori_loop(0, blk // _UNROLL, body, 0)
        out_copy(slab, i, slot).start()

    par = jax.lax.rem(i, 2)

    @pl.when(par == 0)
    def _():
        run(slab_a, smem_a, smem_b, 0)

    @pl.when(par == 1)
    def _():
        run(slab_b, smem_b, smem_a, 1)

    @pl.when(i == nb - 1)
    def _():
        out_copy(slab_a, 0, par).wait()
        if nb > 1:
            out_copy(slab_a, 0, 1 - par).wait()


def kernel(x, table):
    B, T = x.shape
    V, D = table.shape
    N = B * T
    ids = x.reshape(N).astype(jnp.int32)

    blk = _BLK
    n_pad = (-N) % blk
    if n_pad:
        ids = jnp.pad(ids, (0, n_pad))
    n_tot = N + n_pad
    nb = n_tot // blk

    ids3 = ids.reshape(nb, 1, blk)
    table3 = table.reshape(V, 1, D)

    table_bytes = V * D * jnp.dtype(table.dtype).itemsize
    vmem_limit = int(table_bytes + 2 * blk * D * 4 + n_tot * 4 + (6 << 20))

    out = pl.pallas_call(
        _gather_kernel,
        out_shape=jax.ShapeDtypeStruct((n_tot, D), table.dtype),
        grid_spec=pltpu.PrefetchScalarGridSpec(
            num_scalar_prefetch=0,
            grid=(nb,),
            in_specs=[
                pl.BlockSpec((nb, 1, blk), lambda i: (0, 0, 0),
                             pipeline_mode=pl.Buffered(1)),
                pl.BlockSpec((V, 1, D), lambda i: (0, 0, 0),
                             pipeline_mode=pl.Buffered(1)),
            ],
            out_specs=pl.BlockSpec(memory_space=pl.ANY),
            scratch_shapes=[
                pltpu.SMEM((1, blk), jnp.int32),
                pltpu.SMEM((1, blk), jnp.int32),
                pltpu.VMEM((blk, 1, D), table.dtype),
                pltpu.VMEM((blk, 1, D), table.dtype),
                pltpu.SemaphoreType.DMA((2,)),
                pltpu.SemaphoreType.DMA((2,)),
            ],
        ),
        compiler_params=pltpu.CompilerParams(
            dimension_semantics=("arbitrary",),
            vmem_limit_bytes=vmem_limit,
        ),
        cost_estimate=pl.CostEstimate(
            flops=0,
            transcendentals=0,
            bytes_accessed=table_bytes + n_tot * D * 4 + n_tot * 4,
        ),
    )(ids3, table3)

    if n_pad:
        out = out[:N]
    return out.reshape(B, T, D)


# speedup vs baseline: 1.0202x; 1.0202x over previous
import jax
import jax.numpy as jnp
from jax.experimental import pallas as pl
from jax.experimental.pallas import tpu as pltpu

_BLK = 8192
_UNROLL = 256


def _gather_kernel(ids_ref, table_ref, out_hbm, smem_a, smem_b, slab_a,
                   slab_b, sem_ids, sems):
    i = pl.program_id(0)
    nb = pl.num_programs(0)
    blk = slab_a.shape[0]

    def out_copy(slab, step, slot):
        return pltpu.make_async_copy(
            slab.at[:, 0],
            out_hbm.at[pl.ds(step * blk, blk), :],
            sems.at[slot])

    def ids_copy(step, smem, slot):
        return pltpu.make_async_copy(
            ids_ref.at[step, 0], smem.at[0], sem_ids.at[slot])

    @pl.when(i == 0)
    def _():
        ids_copy(0, smem_a, 0).start()

    def run(slab, smem, smem_next, slot):
        @pl.when(i + 1 < nb)
        def _():
            ids_copy(i + 1, smem_next, 1 - slot).start()

        ids_copy(i, smem, slot).wait()

        @pl.when(i >= 2)
        def _():
            out_copy(slab, i - 2, slot).wait()

        def body(o, _):
            base = o * _UNROLL
            for k in range(_UNROLL):
                slab[base + k, 0] = table_ref[smem[0, base + k], 0]
            return 0

        jax.lax.fori_loop(0, blk // _UNROLL, body, 0)
        out_copy(slab, i, slot).start()

    par = jax.lax.rem(i, 2)

    @pl.when(par == 0)
    def _():
        run(slab_a, smem_a, smem_b, 0)

    @pl.when(par == 1)
    def _():
        run(slab_b, smem_b, smem_a, 1)

    @pl.when(i == nb - 1)
    def _():
        out_copy(slab_a, 0, par).wait()
        if nb > 1:
            out_copy(slab_a, 0, 1 - par).wait()


def kernel(x, table):
    B, T = x.shape
    V, D = table.shape
    N = B * T
    ids = x.reshape(N).astype(jnp.int32)

    blk = _BLK
    n_pad = (-N) % blk
    if n_pad:
        ids = jnp.pad(ids, (0, n_pad))
    n_tot = N + n_pad
    nb = n_tot // blk

    ids3 = ids.reshape(nb, 1, blk)
    table3 = table.reshape(V, 1, D)

    table_bytes = V * D * jnp.dtype(table.dtype).itemsize
    vmem_limit = int(table_bytes + 2 * blk * D * 4 + n_tot * 4 + (6 << 20))

    out = pl.pallas_call(
        _gather_kernel,
        out_shape=jax.ShapeDtypeStruct((n_tot, D), table.dtype),
        grid_spec=pltpu.PrefetchScalarGridSpec(
            num_scalar_prefetch=0,
            grid=(nb,),
            in_specs=[
                pl.BlockSpec((nb, 1, blk), lambda i: (0, 0, 0),
                             pipeline_mode=pl.Buffered(1)),
                pl.BlockSpec((V, 1, D), lambda i: (0, 0, 0),
                             pipeline_mode=pl.Buffered(1)),
            ],
            out_specs=pl.BlockSpec(memory_space=pl.ANY),
            scratch_shapes=[
                pltpu.SMEM((1, blk), jnp.int32),
                pltpu.SMEM((1, blk), jnp.int32),
                pltpu.VMEM((blk, 1, D), table.dtype),
                pltpu.VMEM((blk, 1, D), table.dtype),
                pltpu.SemaphoreType.DMA((2,)),
                pltpu.SemaphoreType.DMA((2,)),
            ],
        ),
        compiler_params=pltpu.CompilerParams(
            dimension_semantics=("arbitrary",),
            vmem_limit_bytes=vmem_limit,
        ),
        cost_estimate=pl.CostEstimate(
            flops=0,
            transcendentals=0,
            bytes_accessed=table_bytes + n_tot * D * 4 + n_tot * 4,
        ),
    )(ids3, table3)

    if n_pad:
        out = out[:N]
    return out.reshape(B, T, D)


# speedup vs baseline: 1.0286x; 1.0082x over previous
import jax
import jax.numpy as jnp
from jax.experimental import pallas as pl
from jax.experimental.pallas import tpu as pltpu

_BLK = 8192
_UNROLL = 512


def _gather_kernel(ids_ref, table_ref, out_hbm, smem_a, smem_b, slab_a,
                   slab_b, sem_ids, sems):
    i = pl.program_id(0)
    nb = pl.num_programs(0)
    blk = slab_a.shape[0]

    def out_copy(slab, step, slot):
        return pltpu.make_async_copy(
            slab.at[:, 0],
            out_hbm.at[pl.ds(step * blk, blk), :],
            sems.at[slot])

    def ids_copy(step, smem, slot):
        return pltpu.make_async_copy(
            ids_ref.at[step, 0], smem.at[0], sem_ids.at[slot])

    @pl.when(i == 0)
    def _():
        ids_copy(0, smem_a, 0).start()

    def run(slab, smem, smem_next, slot):
        @pl.when(i + 1 < nb)
        def _():
            ids_copy(i + 1, smem_next, 1 - slot).start()

        ids_copy(i, smem, slot).wait()

        @pl.when(i >= 2)
        def _():
            out_copy(slab, i - 2, slot).wait()

        def body(o, _):
            base = o * _UNROLL
            for k in range(_UNROLL):
                slab[base + k, 0] = table_ref[smem[0, base + k], 0]
            return 0

        jax.lax.fori_loop(0, blk // _UNROLL, body, 0)
        out_copy(slab, i, slot).start()

    par = jax.lax.rem(i, 2)

    @pl.when(par == 0)
    def _():
        run(slab_a, smem_a, smem_b, 0)

    @pl.when(par == 1)
    def _():
        run(slab_b, smem_b, smem_a, 1)

    @pl.when(i == nb - 1)
    def _():
        out_copy(slab_a, 0, par).wait()
        if nb > 1:
            out_copy(slab_a, 0, 1 - par).wait()


def kernel(x, table):
    B, T = x.shape
    V, D = table.shape
    N = B * T
    ids = x.reshape(N).astype(jnp.int32)

    blk = _BLK
    n_pad = (-N) % blk
    if n_pad:
        ids = jnp.pad(ids, (0, n_pad))
    n_tot = N + n_pad
    nb = n_tot // blk

    ids3 = ids.reshape(nb, 1, blk)
    table3 = table.reshape(V, 1, D)

    table_bytes = V * D * jnp.dtype(table.dtype).itemsize
    vmem_limit = int(table_bytes + 2 * blk * D * 4 + n_tot * 4 + (6 << 20))

    out = pl.pallas_call(
        _gather_kernel,
        out_shape=jax.ShapeDtypeStruct((n_tot, D), table.dtype),
        grid_spec=pltpu.PrefetchScalarGridSpec(
            num_scalar_prefetch=0,
            grid=(nb,),
            in_specs=[
                pl.BlockSpec((nb, 1, blk), lambda i: (0, 0, 0),
                             pipeline_mode=pl.Buffered(1)),
                pl.BlockSpec((V, 1, D), lambda i: (0, 0, 0),
                             pipeline_mode=pl.Buffered(1)),
            ],
            out_specs=pl.BlockSpec(memory_space=pl.ANY),
            scratch_shapes=[
                pltpu.SMEM((1, blk), jnp.int32),
                pltpu.SMEM((1, blk), jnp.int32),
                pltpu.VMEM((blk, 1, D), table.dtype),
                pltpu.VMEM((blk, 1, D), table.dtype),
                pltpu.SemaphoreType.DMA((2,)),
                pltpu.SemaphoreType.DMA((2,)),
            ],
        ),
        compiler_params=pltpu.CompilerParams(
            dimension_semantics=("arbitrary",),
            vmem_limit_bytes=vmem_limit,
        ),
        cost_estimate=pl.CostEstimate(
            flops=0,
            transcendentals=0,
            bytes_accessed=table_bytes + n_tot * D * 4 + n_tot * 4,
        ),
    )(ids3, table3)

    if n_pad:
        out = out[:N]
    return out.reshape(B, T, D)
